# baseline (speedup 1.0000x reference)
import functools

import jax
import jax.numpy as jnp
from jax import lax
from jax.experimental import pallas as pl
from jax.experimental.pallas import tpu as pltpu
from jax.experimental.pallas import tpu_sc as plsc

NC = 2
NS = 16
NW = NC * NS
LANES = 16

BATCH = 16384
FEAT = 512
RPW = BATCH // NW
CH = 32
NCHUNK = RPW // CH
NPAIR = NCHUNK // 2

_mesh = plsc.VectorSubcoreMesh(
    core_axis_name="c", subcore_axis_name="s", num_cores=NC, num_subcores=NS
)


@functools.partial(
    pl.kernel,
    out_type=jax.ShapeDtypeStruct((NW, LANES), jnp.float32),
    mesh=_mesh,
    scratch_types=[
        pltpu.VMEM((RPW,), jnp.int32),
        pltpu.VMEM((2, CH, FEAT), jnp.float32),
        pltpu.VMEM((2, CH, FEAT), jnp.float32),
        pltpu.VMEM((LANES,), jnp.float32),
        pltpu.SemaphoreType.DMA,
        pltpu.SemaphoreType.DMA,
        pltpu.SemaphoreType.DMA,
        pltpu.SemaphoreType.DMA,
    ],
)
def _center_loss_sc(x_hbm, lab_hbm, cen_hbm, out_hbm,
                    lab_v, x_v, c_v, o_v, sx0, sx1, sc0, sc1):
    wid = lax.axis_index("s") * NC + lax.axis_index("c")
    base = wid * RPW
    pltpu.sync_copy(lab_hbm.at[pl.ds(base, RPW)], lab_v)

    sems = ((sx0, sc0), (sx1, sc1))
    zeros = jnp.zeros((LANES,), jnp.float32)
    iota = lax.iota(jnp.int32, LANES)
    bfly_idx = [(iota ^ sh)[:, None] for sh in (8, 4, 2, 1)]
    gdn = lax.GatherDimensionNumbers(
        offset_dims=(), collapsed_slice_dims=(0,), start_index_map=(0,))

    def lane_sum(v):
        for idx in bfly_idx:
            v = v + lax.gather(v, idx, gdn, (1,),
                               mode=lax.GatherScatterMode.PROMISE_IN_BOUNDS)
        return v

    def issue(g, slot):
        sx, sc = sems[slot]
        pltpu.async_copy(x_hbm.at[pl.ds(base + g * CH, CH)],
                         x_v.at[slot], sx)
        pltpu.async_copy(cen_hbm.at[lab_v.at[pl.ds(g * CH, CH)]],
                         c_v.at[slot], sc)

    def wait(slot):
        sx, sc = sems[slot]
        pltpu.make_async_copy(x_hbm.at[pl.ds(0, CH)], x_v.at[slot], sx).wait()
        pltpu.make_async_copy(cen_hbm.at[pl.ds(0, CH)], c_v.at[slot],
                              sc).wait()

    def compute(slot, total):
        def row_body(row, tot):
            a0 = zeros
            a1 = zeros
            a2 = zeros
            a3 = zeros
            for j in range(FEAT // LANES):
                d = (x_v[slot, row, pl.ds(j * LANES, LANES)]
                     - c_v[slot, row, pl.ds(j * LANES, LANES)])
                sq = d * d
                if j % 4 == 0:
                    a0 = a0 + sq
                elif j % 4 == 1:
                    a1 = a1 + sq
                elif j % 4 == 2:
                    a2 = a2 + sq
                else:
                    a3 = a3 + sq
            dist = lane_sum((a0 + a1) + (a2 + a3))
            dist = jnp.clip(dist, jnp.float32(1e-12), jnp.float32(1e12))
            return tot + dist

        return lax.fori_loop(0, CH, row_body, total, unroll=2)

    issue(0, 0)

    def pair_body(p, total):
        g0 = 2 * p
        wait(0)
        issue(g0 + 1, 1)
        total = compute(0, total)
        wait(1)

        @pl.when(p < NPAIR - 1)
        def _():
            issue(g0 + 2, 0)

        return compute(1, total)

    total = lax.fori_loop(0, NPAIR, pair_body, zeros, unroll=False)
    o_v[...] = total
    pltpu.sync_copy(o_v, out_hbm.at[wid])


def kernel(x, labels, centers):
    partials = _center_loss_sc(x, labels.astype(jnp.int32), centers)
    return jnp.sum(partials) / jnp.float32(x.shape[0] * LANES)


# speedup vs baseline: 1.3376x; 1.3376x over previous
import functools

import jax
import jax.numpy as jnp
from jax import lax
from jax.experimental import pallas as pl
from jax.experimental.pallas import tpu as pltpu
from jax.experimental.pallas import tpu_sc as plsc

NC = 2
NS = 16
NW = NC * NS
LANES = 16

BATCH = 16384
FEAT = 512
RPW = BATCH // NW
CH = 32
NCHUNK = RPW // CH
NPAIR = NCHUNK // 2

_mesh = plsc.VectorSubcoreMesh(
    core_axis_name="c", subcore_axis_name="s", num_cores=NC, num_subcores=NS
)


@functools.partial(
    pl.kernel,
    out_type=jax.ShapeDtypeStruct((NW, LANES), jnp.float32),
    mesh=_mesh,
    scratch_types=[
        pltpu.VMEM((RPW,), jnp.int32),
        pltpu.VMEM((2, CH, FEAT), jnp.float32),
        pltpu.VMEM((2, CH, FEAT), jnp.float32),
        pltpu.VMEM((LANES,), jnp.float32),
        pltpu.SemaphoreType.DMA,
        pltpu.SemaphoreType.DMA,
        pltpu.SemaphoreType.DMA,
        pltpu.SemaphoreType.DMA,
    ],
)
def _center_loss_sc(x_hbm, lab_hbm, cen_hbm, out_hbm,
                    lab_v, x_v, c_v, o_v, sx0, sx1, sc0, sc1):
    wid = lax.axis_index("s") * NC + lax.axis_index("c")
    base = wid * RPW
    pltpu.sync_copy(lab_hbm.at[pl.ds(base, RPW)], lab_v)

    sems = ((sx0, sc0), (sx1, sc1))
    zeros = jnp.zeros((LANES,), jnp.float32)
    iota = lax.iota(jnp.int32, LANES)
    bfly_idx = [(iota ^ sh)[:, None] for sh in (8, 4, 2, 1)]
    gdn = lax.GatherDimensionNumbers(
        offset_dims=(), collapsed_slice_dims=(0,), start_index_map=(0,))

    def lane_sum(v):
        for idx in bfly_idx:
            v = v + lax.gather(v, idx, gdn, (1,),
                               mode=lax.GatherScatterMode.PROMISE_IN_BOUNDS)
        return v

    def issue(g, slot):
        sx, sc = sems[slot]
        pltpu.async_copy(x_hbm.at[pl.ds(base + g * CH, CH)],
                         x_v.at[slot], sx)
        pltpu.async_copy(cen_hbm.at[lab_v.at[pl.ds(g * CH, CH)]],
                         c_v.at[slot], sc)

    def wait(slot):
        sx, sc = sems[slot]
        pltpu.make_async_copy(x_hbm.at[pl.ds(0, CH)], x_v.at[slot], sx).wait()
        pltpu.make_async_copy(cen_hbm.at[pl.ds(0, CH)], c_v.at[slot],
                              sc).wait()

    def compute(slot, total):
        def row_body(row, tot):
            a0 = zeros
            a1 = zeros
            a2 = zeros
            a3 = zeros
            for j in range(FEAT // LANES):
                d = (x_v[slot, row, pl.ds(j * LANES, LANES)]
                     - c_v[slot, row, pl.ds(j * LANES, LANES)])
                sq = d * d
                if j % 4 == 0:
                    a0 = a0 + sq
                elif j % 4 == 1:
                    a1 = a1 + sq
                elif j % 4 == 2:
                    a2 = a2 + sq
                else:
                    a3 = a3 + sq
            dist = lane_sum((a0 + a1) + (a2 + a3))
            dist = jnp.clip(dist, jnp.float32(1e-12), jnp.float32(1e12))
            return tot + dist

        return lax.fori_loop(0, CH, row_body, total, unroll=False)

    issue(0, 0)

    def pair_body(p, total):
        g0 = 2 * p
        wait(0)
        issue(g0 + 1, 1)
        total = compute(0, total)
        wait(1)

        @pl.when(p < NPAIR - 1)
        def _():
            issue(g0 + 2, 0)

        return compute(1, total)

    total = lax.fori_loop(0, NPAIR, pair_body, zeros, unroll=False)
    o_v[...] = total
    pltpu.sync_copy(o_v, out_hbm.at[wid])


def kernel(x, labels, centers):
    partials = _center_loss_sc(x, labels.astype(jnp.int32), centers)
    return jnp.sum(partials) / jnp.float32(x.shape[0] * LANES)
